# Initial kernel scaffold; baseline (speedup 1.0000x reference)
import jax
import jax.numpy as jnp
from jax.experimental import pallas as pl


def kernel(w, log_scale, nf4_codebook):
    raise NotImplementedError("write your pallas kernel here")


# speedup vs baseline: 3.4978x; 3.4978x over previous
import jax
import jax.numpy as jnp
from jax import lax
from jax.experimental import pallas as pl
from jax.experimental.pallas import tpu as pltpu
from jax.experimental.pallas import tpu_sc as plsc

BLOCK = 64
EPS = 1e-06
K = 128
KF = float(K)
NCELL_PAD = 272
CHUNK = 16384
LANES = 16


def _build_sc_call(n_total):
    info = plsc.get_sparse_core_info()
    nworkers = info.num_cores * info.num_subcores
    wpw = n_total // nworkers
    assert n_total % (nworkers * BLOCK) == 0
    nchunk = wpw // CHUNK
    assert wpw % CHUNK == 0

    def body(w_hbm, ls_hbm, cb_hbm, out_hbm,
             in_v, out_v, thr_v, lo_v, hi_v, cb_v, mid_v, ev_v):
        wid = lax.axis_index("s") * info.num_cores + lax.axis_index("c")
        pltpu.sync_copy(cb_hbm, cb_v)
        pltpu.sync_copy(ls_hbm, ev_v)
        iota = lax.broadcasted_iota(jnp.int32, (LANES,), 0)
        c = cb_v[...]
        cnext = plsc.load_gather(cb_v, [jnp.minimum(iota + 1, 15)])
        mid = jnp.where(iota < 15, 0.5 * (c + cnext), jnp.float32(1e30))
        mid_v[...] = mid
        ev = jnp.exp(ev_v[...])
        ev_v[...] = ev

        for t in range(NCELL_PAD // LANES):
            u = t * LANES + iota
            eu = (u.astype(jnp.float32) - KF) * (1.0 / KF)
            n = jnp.zeros((LANES,), jnp.int32)
            for i in range(15):
                n = n + jnp.where(mid[i] < eu, 1, 0)
            sl = pl.ds(t * LANES, LANES)
            thr_v[sl] = plsc.load_gather(mid_v, [n]) * KF
            lo_v[sl] = plsc.load_gather(cb_v, [n])
            hi_v[sl] = plsc.load_gather(cb_v, [jnp.minimum(n + 1, 15)])

        for ch in range(nchunk):
            off = wid * wpw + ch * CHUNK
            pltpu.sync_copy(w_hbm.at[pl.ds(off, CHUNK)], in_v)

            def blk(b, carry):
                base = b * BLOCK
                x0 = in_v[pl.ds(base, LANES)]
                x1 = in_v[pl.ds(base + 16, LANES)]
                x2 = in_v[pl.ds(base + 32, LANES)]
                x3 = in_v[pl.ds(base + 48, LANES)]
                a = jnp.maximum(jnp.maximum(jnp.abs(x0), jnp.abs(x1)),
                                jnp.maximum(jnp.abs(x2), jnp.abs(x3)))
                m = jnp.max(a)
                sv = jnp.maximum(jnp.broadcast_to(m, (LANES,)) * ev,
                                 jnp.float32(EPS))
                inv_k = KF / sv
                for j, x in enumerate((x0, x1, x2, x3)):
                    uf = jnp.clip(x * inv_k, -KF, KF)
                    ui = (uf + KF).astype(jnp.int32)
                    thr = plsc.load_gather(thr_v, [ui])
                    lov = plsc.load_gather(lo_v, [ui])
                    hiv = plsc.load_gather(hi_v, [ui])
                    out_v[pl.ds(base + j * LANES, LANES)] = (
                        jnp.where(uf > thr, hiv, lov) * sv)
                return carry

            lax.fori_loop(0, CHUNK // BLOCK, blk, 0)
            pltpu.sync_copy(out_v, out_hbm.at[pl.ds(off, CHUNK)])

    return pl.kernel(
        body,
        out_type=jax.ShapeDtypeStruct((n_total,), jnp.float32),
        mesh=plsc.VectorSubcoreMesh(core_axis_name="c", subcore_axis_name="s"),
        compiler_params=pltpu.CompilerParams(needs_layout_passes=False),
        scratch_types=[
            pltpu.VMEM((CHUNK,), jnp.float32),
            pltpu.VMEM((CHUNK,), jnp.float32),
            pltpu.VMEM((NCELL_PAD,), jnp.float32),
            pltpu.VMEM((NCELL_PAD,), jnp.float32),
            pltpu.VMEM((NCELL_PAD,), jnp.float32),
            pltpu.VMEM((LANES,), jnp.float32),
            pltpu.VMEM((LANES,), jnp.float32),
            pltpu.VMEM((LANES,), jnp.float32),
        ],
    )


def kernel(w, log_scale, nf4_codebook):
    out_f, in_f = w.shape
    wf = w.reshape(-1)
    ls16 = jnp.broadcast_to(log_scale.astype(jnp.float32), (LANES,))
    cb = nf4_codebook.astype(jnp.float32)
    out = _build_sc_call(wf.shape[0])(wf, ls16, cb)
    return out.reshape(out_f, in_f)
